# baseline (speedup 1.0000x reference)
import math
import functools

import jax
import jax.numpy as jnp
import numpy as np
from jax import lax
from jax.experimental import pallas as pl
from jax.experimental.pallas import tpu as pltpu
from jax.experimental.pallas import tpu_sc as plsc

VOCAB = 100000
D_MODEL = 128
MAX_LEN = 256
BATCH = 1024
SEQ = 200

_NC = 2
_NS = 16
_NW = _NC * _NS
_SEQ_PER_W = BATCH // _NW
_SCALE = jnp.float32(math.sqrt(float(D_MODEL)))

_SPLIT = 104


def _sinusoidal_pe_rows(seq, d_model):
    pos = np.arange(MAX_LEN, dtype=np.float32)[:, None]
    div = np.exp(
        np.arange(0, d_model, 2, dtype=np.float32)
        * np.float32(-math.log(10000.0) / d_model)
    ).astype(np.float32)
    pe = np.zeros((MAX_LEN, d_model), dtype=np.float32)
    pe[:, 0::2] = np.sin(pos * div, dtype=np.float32)
    pe[:, 1::2] = np.cos(pos * div, dtype=np.float32)
    return pe[:seq]


_PE = _sinusoidal_pe_rows(SEQ, D_MODEL)


_NSLOT = 3


def _body(x_hbm, table_hbm, pe_hbm, out_hbm,
          idx_v, rows0, rows1, rows2, pe_v,
          gs0, gs1, gs2, os0, os1, os2):
    wid = lax.axis_index("s") * _NC + lax.axis_index("c")
    seq0 = wid * _SEQ_PER_W

    pe_copy = pltpu.make_async_copy(pe_hbm, pe_v, os0)
    pe_copy.start()
    pltpu.sync_copy(x_hbm.at[pl.ds(seq0 * SEQ, _SEQ_PER_W * SEQ)], idx_v)

    rows = (rows0, rows1, rows2)
    gsem = (gs0, gs1, gs2)
    osem = (os0, os1, os2)

    def gather_copies(i, b):
        off = i * SEQ
        return (
            (table_hbm.at[idx_v.at[pl.ds(off, _SPLIT)]],
             rows[b].at[pl.ds(0, _SPLIT), :], gsem[b]),
            (table_hbm.at[idx_v.at[pl.ds(off + _SPLIT, SEQ - _SPLIT)]],
             rows[b].at[pl.ds(_SPLIT, SEQ - _SPLIT), :], gsem[b]),
        )

    def fire_gather(i, b):
        for c in gather_copies(i, b):
            pltpu.async_copy(*c)

    def wait_gather(i, b):
        for c in gather_copies(i, b):
            pltpu.make_async_copy(*c).wait()

    def compute(b):
        r_ref = rows[b]

        @plsc.parallel_loop(0, SEQ, unroll=1)
        def row_body(r):
            for j in range(D_MODEL // 16):
                sl = pl.ds(j * 16, 16)
                r_ref[r, sl] = r_ref[r, sl] * _SCALE + pe_v[r, sl]

    def fire_out(i, b):
        base = (seq0 + i) * SEQ
        pltpu.async_copy(rows[b], out_hbm.at[pl.ds(base, SEQ), :], osem[b])

    def wait_out(i, b):
        base = (seq0 + i) * SEQ
        pltpu.make_async_copy(rows[b], out_hbm.at[pl.ds(base, SEQ), :],
                              osem[b]).wait()

    def stage(i, s, refill=True):
        wait_gather(i, s)
        compute(s)
        fire_out(i, s)
        wait_out(i - 1, (s + _NSLOT - 1) % _NSLOT)
        if refill:
            fire_gather(i + 2, (s + _NSLOT - 1) % _NSLOT)

    for i in range(_NSLOT):
        fire_gather(i, i)
    pe_copy.wait()
    wait_gather(0, 0)
    compute(0)
    fire_out(0, 0)

    def loop_body(g, c):
        i0 = 1 + _NSLOT * g
        stage(i0, 1)
        stage(i0 + 1, 2)
        stage(i0 + 2, 0)
        return c

    n_triples = (_SEQ_PER_W - 5) // _NSLOT
    lax.fori_loop(0, n_triples, loop_body, 0)
    i = 1 + _NSLOT * n_triples
    stage(i, i % _NSLOT)
    stage(i + 1, (i + 1) % _NSLOT)
    stage(i + 2, (i + 2) % _NSLOT, refill=False)
    stage(i + 3, (i + 3) % _NSLOT, refill=False)
    wait_out(_SEQ_PER_W - 1, (_SEQ_PER_W - 1) % _NSLOT)


@jax.jit
def _embed(x_flat, table, pe):
    mesh = plsc.VectorSubcoreMesh(core_axis_name="c", subcore_axis_name="s")
    f = pl.kernel(
        _body,
        out_type=jax.ShapeDtypeStruct((BATCH * SEQ, D_MODEL), jnp.float32),
        mesh=mesh,
        scratch_types=[
            pltpu.VMEM((_SEQ_PER_W * SEQ,), jnp.int32),
            pltpu.VMEM((SEQ, D_MODEL), jnp.float32),
            pltpu.VMEM((SEQ, D_MODEL), jnp.float32),
            pltpu.VMEM((SEQ, D_MODEL), jnp.float32),
            pltpu.VMEM((SEQ, D_MODEL), jnp.float32),
            pltpu.SemaphoreType.DMA,
            pltpu.SemaphoreType.DMA,
            pltpu.SemaphoreType.DMA,
            pltpu.SemaphoreType.DMA,
            pltpu.SemaphoreType.DMA,
            pltpu.SemaphoreType.DMA,
        ],
        name="input_embedding_sc",
    )
    return f(x_flat, table, pe)


def kernel(x, table):
    x_flat = x.reshape(-1).astype(jnp.int32)
    out = _embed(x_flat, table, _PE)
    return out.reshape(x.shape[0], x.shape[1], D_MODEL)


# speedup vs baseline: 1.0165x; 1.0165x over previous
import math

import jax
import jax.numpy as jnp
import numpy as np
from jax import lax
from jax.experimental import pallas as pl
from jax.experimental.pallas import tpu as pltpu
from jax.experimental.pallas import tpu_sc as plsc

VOCAB = 100000
D_MODEL = 128
MAX_LEN = 256
BATCH = 1024
SEQ = 200

_NC = 2
_NS = 16
_NW = _NC * _NS
_SEQ_PER_W = BATCH // _NW
_SCALE = jnp.float32(math.sqrt(float(D_MODEL)))

_SPLIT = 104
_NSLOT = 6
_NCHUNK = 2 * _SEQ_PER_W
_CHUNK_LEN = (_SPLIT, SEQ - _SPLIT)


def _sinusoidal_pe_rows(seq, d_model):
    pos = np.arange(MAX_LEN, dtype=np.float32)[:, None]
    div = np.exp(
        np.arange(0, d_model, 2, dtype=np.float32)
        * np.float32(-math.log(10000.0) / d_model)
    ).astype(np.float32)
    pe = np.zeros((MAX_LEN, d_model), dtype=np.float32)
    pe[:, 0::2] = np.sin(pos * div, dtype=np.float32)
    pe[:, 1::2] = np.cos(pos * div, dtype=np.float32)
    return pe[:seq]


_PE = _sinusoidal_pe_rows(SEQ, D_MODEL)


def _body(x_hbm, table_hbm, pe_hbm, out_hbm,
          idx_v, r0, r1, r2, r3, r4, r5, pe_v,
          gs0, gs1, gs2, gs3, gs4, gs5,
          os0, os1, os2, os3, os4, os5):
    wid = lax.axis_index("s") * _NC + lax.axis_index("c")
    seq0 = wid * _SEQ_PER_W

    pe_copy = pltpu.make_async_copy(pe_hbm, pe_v, os0)
    pe_copy.start()
    head = _NSLOT // 2 * SEQ
    idx_rest = pltpu.make_async_copy(
        x_hbm.at[pl.ds(seq0 * SEQ + head, _SEQ_PER_W * SEQ - head)],
        idx_v.at[pl.ds(head, _SEQ_PER_W * SEQ - head)], os1)
    idx_rest.start()
    pltpu.sync_copy(x_hbm.at[pl.ds(seq0 * SEQ, head)],
                    idx_v.at[pl.ds(0, head)])

    rows = (r0, r1, r2, r3, r4, r5)
    gsem = (gs0, gs1, gs2, gs3, gs4, gs5)
    osem = (os0, os1, os2, os3, os4, os5)

    def gather_copy(c):
        seq, part, s = c
        off = seq * SEQ + part * _SPLIT
        return (table_hbm.at[idx_v.at[pl.ds(off, _CHUNK_LEN[part])]],
                rows[s], gsem[s])

    def out_copy(c):
        seq, part, s = c
        base = (seq0 + seq) * SEQ + part * _SPLIT
        return (rows[s], out_hbm.at[pl.ds(base, _CHUNK_LEN[part]), :],
                osem[s])

    def compute(c):
        _, part, s = c
        r_ref = rows[s]
        pe_off = part * _SPLIT

        @plsc.parallel_loop(0, _CHUNK_LEN[part], unroll=1)
        def row_body(r):
            for j in range(D_MODEL // 16):
                sl = pl.ds(j * 16, 16)
                r_ref[r, sl] = r_ref[r, sl] * _SCALE + pe_v[pe_off + r, sl]

    def stage(cur, prev=None, nxt=None):
        pltpu.make_async_copy(*gather_copy(cur)).wait()
        compute(cur)
        pltpu.async_copy(*out_copy(cur))
        if prev is not None:
            pltpu.make_async_copy(*out_copy(prev)).wait()
        if nxt is not None:
            pltpu.async_copy(*gather_copy(nxt))

    def chunk_of(h):
        return (h // 2, h % 2, h % _NSLOT)

    for h in range(_NSLOT):
        pltpu.async_copy(*gather_copy(chunk_of(h)))
    pe_copy.wait()
    stage(chunk_of(0))
    idx_rest.wait()

    def loop_body(g, carry):
        sd = 3 * g
        for k in range(_NSLOT):
            cur = (sd + (1 + k) // 2, (1 + k) % 2, (1 + k) % _NSLOT)
            prev = (sd + k // 2, k % 2, k % _NSLOT)
            nxt = (sd + 3 + k // 2, k % 2, k % _NSLOT)
            stage(cur, prev, nxt)
        return carry

    lax.fori_loop(0, 9, loop_body, 0)

    for h in range(55, _NCHUNK):
        nxt = chunk_of(h + 5) if h + 5 < _NCHUNK else None
        stage(chunk_of(h), chunk_of(h - 1), nxt)
    pltpu.make_async_copy(*out_copy(chunk_of(_NCHUNK - 1))).wait()


@jax.jit
def _embed(x_flat, table, pe):
    mesh = plsc.VectorSubcoreMesh(core_axis_name="c", subcore_axis_name="s")
    f = pl.kernel(
        _body,
        out_type=jax.ShapeDtypeStruct((BATCH * SEQ, D_MODEL), jnp.float32),
        mesh=mesh,
        scratch_types=(
            [pltpu.VMEM((_SEQ_PER_W * SEQ,), jnp.int32)]
            + [pltpu.VMEM((_CHUNK_LEN[s % 2], D_MODEL), jnp.float32)
               for s in range(_NSLOT)]
            + [pltpu.VMEM((SEQ, D_MODEL), jnp.float32)]
            + [pltpu.SemaphoreType.DMA] * (2 * _NSLOT)
        ),
        name="input_embedding_sc",
    )
    return f(x_flat, table, pe)


def kernel(x, table):
    x_flat = x.reshape(-1).astype(jnp.int32)
    out = _embed(x_flat, table, _PE)
    return out.reshape(x.shape[0], x.shape[1], D_MODEL)
